# baseline (speedup 1.0000x reference)
import functools

import jax
import jax.numpy as jnp
from jax import lax
from jax.experimental import pallas as pl
from jax.experimental.pallas import tpu as pltpu
from jax.experimental.pallas import tpu_sc as plsc

_VOCAB = 1000000
_E = 64
_B = 4096
_L = 200
_LANES = 16
_NC = 2
_NS = 16
_NW = _NC * _NS
_BPW = _B // _NW
_CHUNKS = ((0, 128), (128, 72))


def _bow_body(data_hbm, len_hbm, table_hbm, out_hbm,
              idx_v, len_v, rows_v, out_v, sem0, sem1):
  wid = lax.axis_index("s") * _NC + lax.axis_index("c")
  base = wid * _BPW

  pltpu.sync_copy(data_hbm.at[pl.ds(base * _L, _BPW * _L)], idx_v)
  pltpu.sync_copy(len_hbm.at[pl.ds(base, _BPW)], len_v.at[pl.ds(0, _BPW)])

  sems = (sem0, sem1)

  def start_gather(b, buf):
    for off, n in _CHUNKS:
      pltpu.async_copy(
          table_hbm.at[idx_v.at[pl.ds(b * _L + off, n)]],
          rows_v.at[buf, pl.ds(off, n)],
          sems[buf])

  def wait_gather(buf):
    pltpu.make_async_copy(
        table_hbm.at[pl.ds(0, _L)], rows_v.at[buf], sems[buf]).wait()

  def compute(b, buf):
    def fold(l, accs):
      a0, a1, a2, a3 = accs
      a0 = a0 + rows_v[buf, l, pl.ds(0, _LANES)]
      a1 = a1 + rows_v[buf, l, pl.ds(_LANES, _LANES)]
      a2 = a2 + rows_v[buf, l, pl.ds(2 * _LANES, _LANES)]
      a3 = a3 + rows_v[buf, l, pl.ds(3 * _LANES, _LANES)]
      return a0, a1, a2, a3

    zero = jnp.zeros((_LANES,), jnp.float32)
    accs = lax.fori_loop(0, _L, fold, (zero, zero, zero, zero))

    lenf = len_v[pl.ds(b, _LANES)][0].astype(jnp.float32)
    recip = jnp.full((_LANES,), 1.0, jnp.float32) / lenf
    for c in range(4):
      out_v[b, pl.ds(c * _LANES, _LANES)] = accs[c] * recip

  start_gather(0, 0)
  start_gather(1, 1)

  def outer(i, carry):
    g = 2 * i
    for j in range(2):
      b = g + j
      wait_gather(j)
      compute(b, j)

      @pl.when(b + 2 < _BPW)
      def _():
        start_gather(b + 2, j)
    return carry

  lax.fori_loop(0, _BPW // 2, outer, 0)

  pltpu.sync_copy(out_v, out_hbm.at[pl.ds(base, _BPW)])


_bow = functools.partial(
    pl.kernel,
    mesh=plsc.VectorSubcoreMesh(core_axis_name="c", subcore_axis_name="s"),
    out_type=jax.ShapeDtypeStruct((_B, _E), jnp.float32),
    scratch_types=[
        pltpu.VMEM((_BPW * _L,), jnp.int32),
        pltpu.VMEM((_BPW + _LANES,), jnp.int32),
        pltpu.VMEM((2, _L, _E), jnp.float32),
        pltpu.VMEM((_BPW, _E), jnp.float32),
        pltpu.SemaphoreType.DMA,
        pltpu.SemaphoreType.DMA,
    ],
    compiler_params=pltpu.CompilerParams(use_tc_tiling_on_sc=False),
)(_bow_body)


@jax.jit
def kernel(data_bl, length_b, table):
  data_flat = data_bl.reshape(_B * _L)
  len_flat = length_b.reshape(_B)
  return _bow(data_flat, len_flat, table)


# speedup vs baseline: 1.0205x; 1.0205x over previous
import functools

import jax
import jax.numpy as jnp
from jax import lax
from jax.experimental import pallas as pl
from jax.experimental.pallas import tpu as pltpu
from jax.experimental.pallas import tpu_sc as plsc

_VOCAB = 1000000
_E = 64
_B = 4096
_L = 200
_LANES = 16
_NC = 2
_NS = 16
_NW = _NC * _NS
_BPW = _B // _NW
_CHUNKS = ((0, 128), (128, 72))


def _bow_body(data_hbm, len_hbm, table_hbm, out_hbm,
              idx_v, len_v, rows_v, out_v, sem0, sem1):
  wid = lax.axis_index("s") * _NC + lax.axis_index("c")
  base = wid * _BPW

  pltpu.sync_copy(data_hbm.at[pl.ds(base * _L, _BPW * _L)], idx_v)
  pltpu.sync_copy(len_hbm.at[pl.ds(base, _BPW)], len_v.at[pl.ds(0, _BPW)])

  sems = (sem0, sem1)

  def start_gather(b, buf):
    for off, n in _CHUNKS:
      pltpu.async_copy(
          table_hbm.at[idx_v.at[pl.ds(b * _L + off, n)]],
          rows_v.at[buf, pl.ds(off, n)],
          sems[buf])

  def wait_gather(buf):
    pltpu.make_async_copy(
        table_hbm.at[pl.ds(0, _L)], rows_v.at[buf], sems[buf]).wait()

  def compute(b, buf):
    def fold(i, accs):
      l = 2 * i
      out = []
      for c in range(4):
        s = pl.ds(c * _LANES, _LANES)
        out.append(accs[c] + rows_v[buf, l, s])
        out.append(accs[c + 4] + rows_v[buf, l + 1, s])
      return (out[0], out[2], out[4], out[6], out[1], out[3], out[5], out[7])

    zero = jnp.zeros((_LANES,), jnp.float32)
    accs = lax.fori_loop(0, _L // 2, fold, (zero,) * 8, unroll=5)
    accs = tuple(accs[c] + accs[c + 4] for c in range(4))

    lenf = len_v[pl.ds(b, _LANES)][0].astype(jnp.float32)
    recip = jnp.full((_LANES,), 1.0, jnp.float32) / lenf
    for c in range(4):
      out_v[b, pl.ds(c * _LANES, _LANES)] = accs[c] * recip

  start_gather(0, 0)
  start_gather(1, 1)

  def outer(i, carry):
    g = 2 * i
    for j in range(2):
      b = g + j
      wait_gather(j)
      compute(b, j)

      @pl.when(b + 2 < _BPW)
      def _():
        start_gather(b + 2, j)
    return carry

  lax.fori_loop(0, _BPW // 2, outer, 0)

  pltpu.sync_copy(out_v, out_hbm.at[pl.ds(base, _BPW)])


_bow = functools.partial(
    pl.kernel,
    mesh=plsc.VectorSubcoreMesh(core_axis_name="c", subcore_axis_name="s"),
    out_type=jax.ShapeDtypeStruct((_B, _E), jnp.float32),
    scratch_types=[
        pltpu.VMEM((_BPW * _L,), jnp.int32),
        pltpu.VMEM((_BPW + _LANES,), jnp.int32),
        pltpu.VMEM((2, _L, _E), jnp.float32),
        pltpu.VMEM((_BPW, _E), jnp.float32),
        pltpu.SemaphoreType.DMA,
        pltpu.SemaphoreType.DMA,
    ],
    compiler_params=pltpu.CompilerParams(use_tc_tiling_on_sc=False),
)(_bow_body)


@jax.jit
def kernel(data_bl, length_b, table):
  data_flat = data_bl.reshape(_B * _L)
  len_flat = length_b.reshape(_B)
  return _bow(data_flat, len_flat, table)


# speedup vs baseline: 1.0797x; 1.0580x over previous
import functools

import jax
import jax.numpy as jnp
from jax import lax
from jax.experimental import pallas as pl
from jax.experimental.pallas import tpu as pltpu
from jax.experimental.pallas import tpu_sc as plsc

_VOCAB = 1000000
_E = 64
_B = 4096
_L = 200
_LANES = 16
_NC = 2
_NS = 16
_NW = _NC * _NS
_BPW = _B // _NW
_G = 2
_GL = _G * _L
_NBUF = 4
_NGROUP = _BPW // _G
_CHUNKS = ((0, 128), (128, 128), (256, 128), (384, 16))


def _bow_body(data_hbm, len_hbm, table_hbm, out_hbm,
              idx_v, len_v, rows_v, outb_v, sem_g, sem_o):
  wid = lax.axis_index("s") * _NC + lax.axis_index("c")
  base = wid * _BPW

  pltpu.sync_copy(data_hbm.at[pl.ds(base * _L, _BPW * _L)], idx_v)
  pltpu.sync_copy(len_hbm.at[pl.ds(base, _BPW)], len_v.at[pl.ds(0, _BPW)])

  def start_gathers(g, buf):
    for off, n in _CHUNKS:
      pltpu.async_copy(
          table_hbm.at[idx_v.at[pl.ds(g * _GL + off, n)]],
          rows_v.at[buf, pl.ds(off, n)],
          sem_g[buf])

  def wait_gathers(buf):
    pltpu.make_async_copy(
        table_hbm.at[pl.ds(0, _GL)], rows_v.at[buf], sem_g[buf]).wait()

  def compute_row(g, buf, r):
    def fold(i, accs):
      l = r * _L + 2 * i
      out = []
      for c in range(4):
        s = pl.ds(c * _LANES, _LANES)
        out.append(accs[c] + rows_v[buf, l, s])
        out.append(accs[c + 4] + rows_v[buf, l + 1, s])
      return (out[0], out[2], out[4], out[6], out[1], out[3], out[5], out[7])

    zero = jnp.zeros((_LANES,), jnp.float32)
    accs = lax.fori_loop(0, _L // 2, fold, (zero,) * 8, unroll=5)

    b = g * _G + r
    lenf = len_v[pl.ds(b, _LANES)][0].astype(jnp.float32)
    recip = jnp.full((_LANES,), 1.0, jnp.float32) / lenf
    for c in range(4):
      outb_v[buf, r, pl.ds(c * _LANES, _LANES)] = (accs[c] + accs[c + 4]) * recip

  for j in range(_NBUF - 1):
    start_gathers(j, j)

  def step(g, buf):
    wait_gathers(buf)

    @pl.when(g >= _NBUF)
    def _():
      pltpu.make_async_copy(
          outb_v.at[buf], out_hbm.at[pl.ds(0, _G)], sem_o[buf]).wait()

    for r in range(_G):
      compute_row(g, buf, r)
    pltpu.async_copy(
        outb_v.at[buf], out_hbm.at[pl.ds(base + g * _G, _G)], sem_o[buf])

    @pl.when(g + _NBUF - 1 < _NGROUP)
    def _():
      start_gathers(g + _NBUF - 1, (buf + _NBUF - 1) % _NBUF)

  def outer(i, carry):
    for j in range(_NBUF):
      step(_NBUF * i + j, j)
    return carry

  lax.fori_loop(0, _NGROUP // _NBUF, outer, 0)

  for j in range(_NBUF):
    pltpu.make_async_copy(
        outb_v.at[j], out_hbm.at[pl.ds(0, _G)], sem_o[j]).wait()


_bow = functools.partial(
    pl.kernel,
    mesh=plsc.VectorSubcoreMesh(core_axis_name="c", subcore_axis_name="s"),
    out_type=jax.ShapeDtypeStruct((_B, _E), jnp.float32),
    scratch_types=[
        pltpu.VMEM((_BPW * _L,), jnp.int32),
        pltpu.VMEM((_BPW + _LANES,), jnp.int32),
        pltpu.VMEM((_NBUF, _GL, _E), jnp.float32),
        pltpu.VMEM((_NBUF, _G, _E), jnp.float32),
        [pltpu.SemaphoreType.DMA] * _NBUF,
        [pltpu.SemaphoreType.DMA] * _NBUF,
    ],
    compiler_params=pltpu.CompilerParams(use_tc_tiling_on_sc=False),
)(_bow_body)


@jax.jit
def kernel(data_bl, length_b, table):
  data_flat = data_bl.reshape(_B * _L)
  len_flat = length_b.reshape(_B)
  return _bow(data_flat, len_flat, table)
